# baseline (speedup 1.0000x reference)
import functools

import jax
import jax.numpy as jnp
from jax import lax
from jax.experimental import pallas as pl
from jax.experimental.pallas import tpu as pltpu
from jax.experimental.pallas import tpu_sc as plsc

NN = 100000
EE = 1600000
NP = 102400
TR = 32768
TRBITS = 15
NRANGE = 4
NSUB = 16
NCORE = 2
ROWS_PER_SUB = NP // NSUB
TR_SUB = TR // NSUB
LAST_ROWS = NP - 3 * TR
RB = 6400


def _fill_f32(ref, n, value):
  def body(i, _):
    ref[pl.ds(i * 16, 16)] = jnp.full((16,), value, jnp.float32)
    return 0
  lax.fori_loop(0, n // 16, body, 0)


def _fill_rows_f32(ref, rows, value):
  def body(i, _):
    ref[i, :] = jnp.full((16,), value, jnp.float32)
    return 0
  lax.fori_loop(0, rows, body, 0)


def _sc_degree(dst):
  B = 10000
  epw = EE // (NCORE * NSUB)
  nblk = epw // B
  mesh = plsc.VectorSubcoreMesh(core_axis_name="c", subcore_axis_name="s")

  @functools.partial(
      pl.kernel,
      compiler_params=pltpu.CompilerParams(use_tc_tiling_on_sc=False),
      out_type=jax.ShapeDtypeStruct((NCORE, NP), jnp.float32),
      mesh=mesh,
      scratch_types=[
          pltpu.VMEM((B,), jnp.int32),
          pltpu.VMEM((B,), jnp.float32),
          pltpu.VMEM((ROWS_PER_SUB,), jnp.float32),
          pltpu.VMEM_SHARED((NP,), jnp.float32),
      ],
  )
  def deg_kernel(dst_hbm, out_hbm, idx_v, ones_v, zero_v, acc):
    c = lax.axis_index("c")
    s = lax.axis_index("s")
    wid = c * NSUB + s
    _fill_f32(ones_v, B, 1.0)
    _fill_f32(zero_v, ROWS_PER_SUB, 0.0)

    pltpu.sync_copy(zero_v, acc.at[pl.ds(s * ROWS_PER_SUB, ROWS_PER_SUB)])
    plsc.subcore_barrier()
    base = wid * epw

    def body(i, _):
      pltpu.sync_copy(dst_hbm.at[pl.ds(base + i * B, B)], idx_v)
      pltpu.sync_copy(ones_v, acc.at[idx_v], add=True)
      return 0

    lax.fori_loop(0, nblk, body, 0)
    plsc.subcore_barrier()
    sl = pl.ds(s * ROWS_PER_SUB, ROWS_PER_SUB)
    pltpu.sync_copy(acc.at[sl], out_hbm.at[c].at[sl])

  return deg_kernel(dst)


CAPR = (272000, 272000, 272000, 24000)
OFFR = (0, 272000, 544000, 816000)
REG = 840000
ARR = 844288


def _tc_positions(dst2d):
  EB = 12800
  EC = 125
  BLK = 1600

  def body(d_ref, tpos_ref, dadj_ref, carry_ref):
    i = pl.program_id(0)

    @pl.when(jnp.logical_or(i == 0, i == 4))
    def _():
      for r in range(NRANGE):
        carry_ref[r] = 0

    dv = d_ref[...]
    rid = lax.shift_right_logical(dv, TRBITS)
    dadj_ref[...] = lax.bitwise_and(dv, TR - 1)
    trash = REG + lax.broadcasted_iota(jnp.int32, (BLK, EC), 1) * 8
    ui = lax.broadcasted_iota(jnp.int32, (EC, EC), 0)
    uj = lax.broadcasted_iota(jnp.int32, (EC, EC), 1)
    ucum = (ui <= uj).astype(jnp.float32)
    li = lax.broadcasted_iota(jnp.int32, (BLK, BLK), 0)
    lj = lax.broadcasted_iota(jnp.int32, (BLK, BLK), 1)
    lcum = (lj < li).astype(jnp.float32)
    tpos = jnp.zeros((BLK, EC), jnp.int32)
    for r in range(NRANGE):
      ok = (rid == r).astype(jnp.float32)
      inc = jnp.dot(ok, ucum, preferred_element_type=jnp.float32)
      rowtot = inc[:, EC - 1:EC]
      rowoff = jnp.dot(lcum, rowtot, preferred_element_type=jnp.float32)
      pos = carry_ref[r].astype(jnp.float32) + rowoff + inc - 1.0
      posi = pos.astype(jnp.int32)
      slot = jnp.where(posi < CAPR[r], OFFR[r] + posi, trash)
      tpos = jnp.where(ok > 0.5, slot, tpos)
      carry_ref[r] = carry_ref[r] + jnp.sum(ok).astype(jnp.int32)
    tpos_ref[...] = tpos

  return pl.pallas_call(
      body,
      grid=(EB // BLK,),
      in_specs=[
          pl.BlockSpec((BLK, EC), lambda i: (i, 0)),
      ],
      out_specs=[
          pl.BlockSpec((BLK, EC), lambda i: (i, 0)),
          pl.BlockSpec((BLK, EC), lambda i: (i, 0)),
      ],
      out_shape=[
          jax.ShapeDtypeStruct((EB, EC), jnp.int32),
          jax.ShapeDtypeStruct((EB, EC), jnp.int32),
      ],
      scratch_shapes=[pltpu.SMEM((NRANGE,), jnp.int32)],
  )(dst2d)


def _sc_build_compact(src, dadj, tpos):
  B = 2000
  epw = EE // (NCORE * NSUB)
  nblk = epw // B
  FILL = ARR // NSUB
  mesh = plsc.VectorSubcoreMesh(core_axis_name="c", subcore_axis_name="s")

  @functools.partial(
      pl.kernel,
      compiler_params=pltpu.CompilerParams(use_tc_tiling_on_sc=False),
      out_type=[
          jax.ShapeDtypeStruct((NCORE, ARR), jnp.int32),
          jax.ShapeDtypeStruct((NCORE, ARR), jnp.int32),
      ],
      mesh=mesh,
      scratch_types=[
          pltpu.VMEM((FILL,), jnp.int32),
          pltpu.VMEM((B,), jnp.int32),
          pltpu.VMEM((B,), jnp.int32),
          pltpu.VMEM((B,), jnp.int32),
          pltpu.SemaphoreType.DMA,
      ],
  )
  def build_kernel(src_hbm, dadj_hbm, tpos_hbm, outs_hbm, outd_hbm,
                   fillv, sv, dv, tv, sem):
    c = lax.axis_index("c")
    s = lax.axis_index("s")
    wid = c * NSUB + s

    def fill_body(i, _):
      fillv[pl.ds(i * 16, 16)] = jnp.full((16,), -1, jnp.int32)
      return 0

    lax.fori_loop(0, FILL // 16, fill_body, 0)
    sl = pl.ds(s * FILL, FILL)
    pltpu.sync_copy(fillv, outs_hbm.at[c].at[sl])
    pltpu.sync_copy(fillv, outd_hbm.at[c].at[sl])
    plsc.subcore_barrier()

    base = wid * epw

    def body(i, _):
      off = base + i * B
      pltpu.sync_copy(src_hbm.at[pl.ds(off, B)], sv)
      pltpu.sync_copy(dadj_hbm.at[pl.ds(off, B)], dv)
      pltpu.sync_copy(tpos_hbm.at[pl.ds(off, B)], tv)
      pltpu.async_copy(sv, outs_hbm.at[c].at[tv], sem).wait()
      pltpu.async_copy(dv, outd_hbm.at[c].at[tv], sem).wait()
      return 0

    lax.fori_loop(0, nblk, body, 0)

  return build_kernel(src, dadj, tpos)


@functools.cache
def _agg16_kernel():
  B = 2000
  mesh = plsc.VectorSubcoreMesh(core_axis_name="c", subcore_axis_name="s")

  @functools.partial(
      pl.kernel,
      compiler_params=pltpu.CompilerParams(use_tc_tiling_on_sc=False),
      out_type=jax.ShapeDtypeStruct((NCORE, NP, 16), jnp.float32),
      mesh=mesh,
      scratch_types=[
          pltpu.VMEM((B,), jnp.int32),
          pltpu.VMEM((B,), jnp.int32),
          pltpu.VMEM((B,), jnp.int32),
          pltpu.VMEM((B,), jnp.int32),
          pltpu.VMEM((B, 16), jnp.float32),
          pltpu.VMEM((B, 16), jnp.float32),
          pltpu.VMEM((512, 16), jnp.float32),
          pltpu.VMEM_SHARED((TR, 16), jnp.float32),
          pltpu.SemaphoreType.DMA,
          pltpu.SemaphoreType.DMA,
          pltpu.SemaphoreType.DMA,
      ],
  )
  def agg_kernel(souts_hbm, douts_hbm, z_hbm, out_hbm, sidx0, didx0, sidx1,
                 didx1, rows0, rows1, zbuf, acc, semg, sems0, sems1):
    c = lax.axis_index("c")
    s = lax.axis_index("s")
    wid = c * NSUB + s
    _fill_rows_f32(zbuf, 512, 0.0)

    def scat_desc(rows_v, didx_v, sem):
      return pltpu.make_async_copy(
          rows_v, acc.at[plsc.Indices(didx_v, ignored_value=-1)], sem)

    for r in range(NRANGE):
      rblk = CAPR[r] // B
      lo_blk = (wid * rblk) // (NCORE * NSUB)
      hi_blk = ((wid + 1) * rblk) // (NCORE * NSUB)
      for t in range(4):
        pltpu.sync_copy(zbuf, acc.at[pl.ds(s * TR_SUB + t * 512, 512)])
      plsc.subcore_barrier()

      for tset in range(NCORE):
        def chain(sidx_v, didx_v, rows_v, sem_s, i, started):
          @pl.when(started)
          def _():
            scat_desc(rows_v, didx_v, sem_s).wait()

          off = OFFR[r] + i * B
          pltpu.sync_copy(souts_hbm.at[tset].at[pl.ds(off, B)], sidx_v)
          pltpu.sync_copy(douts_hbm.at[tset].at[pl.ds(off, B)], didx_v)
          pltpu.async_copy(
              z_hbm.at[plsc.Indices(sidx_v, ignored_value=-1)], rows_v, semg
          ).wait()
          scat_desc(rows_v, didx_v, sem_s).start(add=True)

        def body(i, _):
          j = i - lo_blk

          @pl.when(j % 2 == 0)
          def _():
            chain(sidx0, didx0, rows0, sems0, i, j >= 2)

          @pl.when(j % 2 == 1)
          def _():
            chain(sidx1, didx1, rows1, sems1, i, j >= 2)

          return 0

        lax.fori_loop(lo_blk, hi_blk, body, 0)
        nw = hi_blk - lo_blk

        @pl.when(nw >= 2)
        def _():
          scat_desc(rows0, didx0, sems0).wait()
          scat_desc(rows1, didx1, sems1).wait()

        @pl.when(nw == 1)
        def _():
          scat_desc(rows0, didx0, sems0).wait()

        del chain, body

      plsc.subcore_barrier()
      nrows = TR_SUB if r < NRANGE - 1 else LAST_ROWS // NSUB
      pltpu.sync_copy(
          acc.at[pl.ds(s * nrows, nrows)],
          out_hbm.at[c].at[pl.ds(r * TR + s * nrows, nrows)])
      plsc.subcore_barrier()

  return agg_kernel


def _sc_agg16(souts, douts, z):
  return _agg16_kernel()(souts, douts, z)


def _tc_prep(degp2, xp):
  grid = NP // RB

  def body(deg_ref, x_ref, z1_ref):
    dcol = lax.rsqrt(deg_ref[0] + deg_ref[1] + 1.0)
    z1 = x_ref[...] * dcol
    z1_ref[...] = jnp.concatenate(
        [z1, jnp.zeros((RB, 8), jnp.float32)], axis=1)

  return pl.pallas_call(
      body,
      grid=(grid,),
      in_specs=[
          pl.BlockSpec((2, RB, 1), lambda i: (0, i, 0)),
          pl.BlockSpec((RB, 8), lambda i: (i, 0)),
      ],
      out_specs=pl.BlockSpec((RB, 16), lambda i: (i, 0)),
      out_shape=jax.ShapeDtypeStruct((NP, 16), jnp.float32),
  )(degp2, xp)


def _tc_layer1(agg1p, z1, degp2, W1, b1):
  grid = NP // RB

  def body(agg_ref, z1_ref, deg_ref, w_ref, b_ref, out_ref):
    dcol = lax.rsqrt(deg_ref[0] + deg_ref[1] + 1.0)
    a1 = (agg_ref[0] + agg_ref[1] + z1_ref[...]) * dcol
    h = jnp.dot(a1[:, :8], w_ref[...], preferred_element_type=jnp.float32)
    h = jnp.maximum(h + b_ref[...], 0.0)
    out_ref[...] = h * dcol

  return pl.pallas_call(
      body,
      grid=(grid,),
      in_specs=[
          pl.BlockSpec((2, RB, 16), lambda i: (0, i, 0)),
          pl.BlockSpec((RB, 16), lambda i: (i, 0)),
          pl.BlockSpec((2, RB, 1), lambda i: (0, i, 0)),
          pl.BlockSpec((8, 64), lambda i: (0, 0)),
          pl.BlockSpec((1, 64), lambda i: (0, 0)),
      ],
      out_specs=pl.BlockSpec((RB, 64), lambda i: (i, 0)),
      out_shape=jax.ShapeDtypeStruct((NP, 64), jnp.float32),
  )(agg1p, z1, degp2, W1, b1)


def _tc_final(agg2cat, z2, degp2, W2, b2, Wfc, bfc):
  grid = NP // RB

  def body(agg_ref, z2_ref, deg_ref, w_ref, b_ref, wfc_ref, bfc_ref,
           out_ref, acc_ref):
    i = pl.program_id(0)

    @pl.when(i == 0)
    def _():
      acc_ref[...] = jnp.zeros_like(acc_ref)

    dcol = lax.rsqrt(deg_ref[0] + deg_ref[1] + 1.0)
    a2 = (agg_ref[0] + agg_ref[1] + z2_ref[...]) * dcol
    h2 = jnp.dot(a2, w_ref[...], preferred_element_type=jnp.float32)
    h2 = jnp.maximum(h2 + b_ref[...], 0.0)
    rowid = i * RB + lax.broadcasted_iota(jnp.int32, (RB, 1), 0)
    h2 = jnp.where(rowid < NN, h2, 0.0)
    acc_ref[...] += jnp.sum(h2, axis=0, keepdims=True)

    @pl.when(i == grid - 1)
    def _():
      g = acc_ref[...] / jnp.float32(NN)
      out_ref[...] = jnp.dot(
          g, wfc_ref[...], preferred_element_type=jnp.float32) + bfc_ref[...]

  return pl.pallas_call(
      body,
      grid=(grid,),
      in_specs=[
          pl.BlockSpec((2, RB, 64), lambda i: (0, i, 0)),
          pl.BlockSpec((RB, 64), lambda i: (i, 0)),
          pl.BlockSpec((2, RB, 1), lambda i: (0, i, 0)),
          pl.BlockSpec((64, 128), lambda i: (0, 0)),
          pl.BlockSpec((1, 128), lambda i: (0, 0)),
          pl.BlockSpec((128, 1), lambda i: (0, 0)),
          pl.BlockSpec((1, 1), lambda i: (0, 0)),
      ],
      out_specs=pl.BlockSpec((1, 1), lambda i: (0, 0)),
      out_shape=jax.ShapeDtypeStruct((1, 1), jnp.float32),
      scratch_shapes=[pltpu.VMEM((1, 128), jnp.float32)],
  )(agg2cat, z2, degp2, W2, b2, Wfc, bfc)


def kernel(x, edge_index, W1, b1, W2, b2, Wfc, bfc):
  src = edge_index[0]
  dst = edge_index[1]
  tpos2d, dadj2d = _tc_positions(dst.reshape(12800, 125))
  souts, douts = _sc_build_compact(src, dadj2d.reshape(EE),
                                   tpos2d.reshape(EE))
  xp = jnp.pad(x, ((0, NP - NN), (0, 0)))

  degp = _sc_degree(dst)
  degp2 = degp.reshape(2, NP, 1)
  z1 = _tc_prep(degp2, xp)
  agg1p = _sc_agg16(souts, douts, z1)
  z2 = _tc_layer1(agg1p, z1, degp2, W1.astype(jnp.float32),
                  b1.reshape(1, 64))
  z2c = [z2[:, 16 * k:16 * (k + 1)] for k in range(4)]
  agg2p = [_sc_agg16(souts, douts, zc) for zc in z2c]
  agg2cat = jnp.concatenate(agg2p, axis=2)
  out = _tc_final(agg2cat, z2, degp2, W2.astype(jnp.float32),
                  b2.reshape(1, 128), Wfc, bfc.reshape(1, 1))
  return out.reshape((1,))


# speedup vs baseline: 2.4092x; 2.4092x over previous
import functools

import jax
import jax.numpy as jnp
from jax import lax
from jax.experimental import pallas as pl
from jax.experimental.pallas import tpu as pltpu
from jax.experimental.pallas import tpu_sc as plsc

NN = 100000
EE = 1600000
NP = 102400
TR = 32768
TRBITS = 15
NRANGE = 4
NSUB = 16
NCORE = 2
ROWS_PER_SUB = NP // NSUB
TR_SUB = TR // NSUB
LAST_ROWS = NP - 3 * TR
RB = 6400


def _fill_f32(ref, n, value):
  def body(i, _):
    ref[pl.ds(i * 16, 16)] = jnp.full((16,), value, jnp.float32)
    return 0
  lax.fori_loop(0, n // 16, body, 0)


def _fill_rows_f32(ref, rows, value):
  def body(i, _):
    ref[i, :] = jnp.full((16,), value, jnp.float32)
    return 0
  lax.fori_loop(0, rows, body, 0)


def _sc_degree(dst):
  B = 10000
  epw = EE // (NCORE * NSUB)
  nblk = epw // B
  mesh = plsc.VectorSubcoreMesh(core_axis_name="c", subcore_axis_name="s")

  @functools.partial(
      pl.kernel,
      compiler_params=pltpu.CompilerParams(use_tc_tiling_on_sc=False),
      out_type=jax.ShapeDtypeStruct((NCORE, NP), jnp.float32),
      mesh=mesh,
      scratch_types=[
          pltpu.VMEM((B,), jnp.int32),
          pltpu.VMEM((B,), jnp.float32),
          pltpu.VMEM((ROWS_PER_SUB,), jnp.float32),
          pltpu.VMEM_SHARED((NP,), jnp.float32),
      ],
  )
  def deg_kernel(dst_hbm, out_hbm, idx_v, ones_v, zero_v, acc):
    c = lax.axis_index("c")
    s = lax.axis_index("s")
    wid = c * NSUB + s
    _fill_f32(ones_v, B, 1.0)
    _fill_f32(zero_v, ROWS_PER_SUB, 0.0)

    pltpu.sync_copy(zero_v, acc.at[pl.ds(s * ROWS_PER_SUB, ROWS_PER_SUB)])
    plsc.subcore_barrier()
    base = wid * epw

    def body(i, _):
      pltpu.sync_copy(dst_hbm.at[pl.ds(base + i * B, B)], idx_v)
      pltpu.sync_copy(ones_v, acc.at[idx_v], add=True)
      return 0

    lax.fori_loop(0, nblk, body, 0)
    plsc.subcore_barrier()
    sl = pl.ds(s * ROWS_PER_SUB, ROWS_PER_SUB)
    pltpu.sync_copy(acc.at[sl], out_hbm.at[c].at[sl])

  return deg_kernel(dst)


def _tc_edge_filter(src2d, dst2d):
  EB = 12800
  EC = 125
  BLK = 1600

  def body(s_ref, d_ref, sf_ref, df_ref):
    sv = s_ref[...]
    dv = d_ref[...]
    rng = lax.shift_right_logical(dv, TRBITS)
    dadj = lax.bitwise_and(dv, TR - 1)
    for r in range(NRANGE):
      ok = rng == r
      sf_ref[r] = jnp.where(ok, sv, -1)
      df_ref[r] = jnp.where(ok, dadj, -1)

  return pl.pallas_call(
      body,
      grid=(EB // BLK,),
      in_specs=[
          pl.BlockSpec((BLK, EC), lambda i: (i, 0)),
          pl.BlockSpec((BLK, EC), lambda i: (i, 0)),
      ],
      out_specs=[
          pl.BlockSpec((NRANGE, BLK, EC), lambda i: (0, i, 0)),
          pl.BlockSpec((NRANGE, BLK, EC), lambda i: (0, i, 0)),
      ],
      out_shape=[
          jax.ShapeDtypeStruct((NRANGE, EB, EC), jnp.int32),
          jax.ShapeDtypeStruct((NRANGE, EB, EC), jnp.int32),
      ],
  )(src2d, dst2d)


@functools.cache
def _agg16_kernel():
  B = 2000
  epw = EE // (NCORE * NSUB)
  nblk = epw // B
  mesh = plsc.VectorSubcoreMesh(core_axis_name="c", subcore_axis_name="s")

  @functools.partial(
      pl.kernel,
      compiler_params=pltpu.CompilerParams(use_tc_tiling_on_sc=False),
      out_type=jax.ShapeDtypeStruct((NCORE, NP, 16), jnp.float32),
      mesh=mesh,
      scratch_types=[
          pltpu.VMEM((B,), jnp.int32),
          pltpu.VMEM((B,), jnp.int32),
          pltpu.VMEM((B,), jnp.int32),
          pltpu.VMEM((B,), jnp.int32),
          pltpu.VMEM((B, 16), jnp.float32),
          pltpu.VMEM((B, 16), jnp.float32),
          pltpu.VMEM((512, 16), jnp.float32),
          pltpu.VMEM_SHARED((TR, 16), jnp.float32),
          pltpu.SemaphoreType.DMA,
          pltpu.SemaphoreType.DMA,
          pltpu.SemaphoreType.DMA,
      ],
  )
  def agg_kernel(sf_hbm, df_hbm, z_hbm, out_hbm, sidx0, didx0, sidx1, didx1,
                 rows0, rows1, zbuf, acc, semg, sems0, sems1):
    c = lax.axis_index("c")
    s = lax.axis_index("s")
    wid = c * NSUB + s
    _fill_rows_f32(zbuf, 512, 0.0)

    def scat_desc(rows_v, didx_v, sem):
      return pltpu.make_async_copy(
          rows_v, acc.at[plsc.Indices(didx_v, ignored_value=-1)], sem)

    for r in range(NRANGE):
      for t in range(4):
        pltpu.sync_copy(zbuf, acc.at[pl.ds(s * TR_SUB + t * 512, 512)])
      plsc.subcore_barrier()
      base = wid * epw

      def chain(sidx_v, didx_v, rows_v, sem_s, i):
        @pl.when(i >= 2)
        def _():
          scat_desc(rows_v, didx_v, sem_s).wait()

        off = base + i * B
        pltpu.sync_copy(sf_hbm.at[r].at[pl.ds(off, B)], sidx_v)
        pltpu.sync_copy(df_hbm.at[r].at[pl.ds(off, B)], didx_v)
        pltpu.async_copy(
            z_hbm.at[plsc.Indices(sidx_v, ignored_value=-1)], rows_v, semg
        ).wait()
        scat_desc(rows_v, didx_v, sem_s).start(add=True)

      def body(i, _):
        @pl.when(i % 2 == 0)
        def _():
          chain(sidx0, didx0, rows0, sems0, i)

        @pl.when(i % 2 == 1)
        def _():
          chain(sidx1, didx1, rows1, sems1, i)

        return 0

      lax.fori_loop(0, nblk, body, 0)
      scat_desc(rows0, didx0, sems0).wait()
      scat_desc(rows1, didx1, sems1).wait()
      plsc.subcore_barrier()
      nrows = TR_SUB if r < NRANGE - 1 else LAST_ROWS // NSUB
      pltpu.sync_copy(
          acc.at[pl.ds(s * nrows, nrows)],
          out_hbm.at[c].at[pl.ds(r * TR + s * nrows, nrows)])
      plsc.subcore_barrier()

  return agg_kernel


def _sc_agg16(sf, df, z):
  return _agg16_kernel()(sf, df, z)


def _tc_prep(degp2, xp):
  grid = NP // RB

  def body(deg_ref, x_ref, z1_ref):
    dcol = lax.rsqrt(deg_ref[0] + deg_ref[1] + 1.0)
    z1 = x_ref[...] * dcol
    z1_ref[...] = jnp.concatenate(
        [z1, jnp.zeros((RB, 8), jnp.float32)], axis=1)

  return pl.pallas_call(
      body,
      grid=(grid,),
      in_specs=[
          pl.BlockSpec((2, RB, 1), lambda i: (0, i, 0)),
          pl.BlockSpec((RB, 8), lambda i: (i, 0)),
      ],
      out_specs=pl.BlockSpec((RB, 16), lambda i: (i, 0)),
      out_shape=jax.ShapeDtypeStruct((NP, 16), jnp.float32),
  )(degp2, xp)


def _tc_layer1(agg1p, z1, degp2, W1, b1):
  grid = NP // RB

  def body(agg_ref, z1_ref, deg_ref, w_ref, b_ref, out_ref):
    dcol = lax.rsqrt(deg_ref[0] + deg_ref[1] + 1.0)
    a1 = (agg_ref[0] + agg_ref[1] + z1_ref[...]) * dcol
    h = jnp.dot(a1[:, :8], w_ref[...], preferred_element_type=jnp.float32)
    h = jnp.maximum(h + b_ref[...], 0.0)
    out_ref[...] = h * dcol

  return pl.pallas_call(
      body,
      grid=(grid,),
      in_specs=[
          pl.BlockSpec((2, RB, 16), lambda i: (0, i, 0)),
          pl.BlockSpec((RB, 16), lambda i: (i, 0)),
          pl.BlockSpec((2, RB, 1), lambda i: (0, i, 0)),
          pl.BlockSpec((8, 64), lambda i: (0, 0)),
          pl.BlockSpec((1, 64), lambda i: (0, 0)),
      ],
      out_specs=pl.BlockSpec((RB, 64), lambda i: (i, 0)),
      out_shape=jax.ShapeDtypeStruct((NP, 64), jnp.float32),
  )(agg1p, z1, degp2, W1, b1)


def _tc_final(agg2cat, z2, degp2, W2, b2, Wfc, bfc):
  grid = NP // RB

  def body(agg_ref, z2_ref, deg_ref, w_ref, b_ref, wfc_ref, bfc_ref,
           out_ref, acc_ref):
    i = pl.program_id(0)

    @pl.when(i == 0)
    def _():
      acc_ref[...] = jnp.zeros_like(acc_ref)

    dcol = lax.rsqrt(deg_ref[0] + deg_ref[1] + 1.0)
    a2 = (agg_ref[0] + agg_ref[1] + z2_ref[...]) * dcol
    h2 = jnp.dot(a2, w_ref[...], preferred_element_type=jnp.float32)
    h2 = jnp.maximum(h2 + b_ref[...], 0.0)
    rowid = i * RB + lax.broadcasted_iota(jnp.int32, (RB, 1), 0)
    h2 = jnp.where(rowid < NN, h2, 0.0)
    acc_ref[...] += jnp.sum(h2, axis=0, keepdims=True)

    @pl.when(i == grid - 1)
    def _():
      g = acc_ref[...] / jnp.float32(NN)
      out_ref[...] = jnp.dot(
          g, wfc_ref[...], preferred_element_type=jnp.float32) + bfc_ref[...]

  return pl.pallas_call(
      body,
      grid=(grid,),
      in_specs=[
          pl.BlockSpec((2, RB, 64), lambda i: (0, i, 0)),
          pl.BlockSpec((RB, 64), lambda i: (i, 0)),
          pl.BlockSpec((2, RB, 1), lambda i: (0, i, 0)),
          pl.BlockSpec((64, 128), lambda i: (0, 0)),
          pl.BlockSpec((1, 128), lambda i: (0, 0)),
          pl.BlockSpec((128, 1), lambda i: (0, 0)),
          pl.BlockSpec((1, 1), lambda i: (0, 0)),
      ],
      out_specs=pl.BlockSpec((1, 1), lambda i: (0, 0)),
      out_shape=jax.ShapeDtypeStruct((1, 1), jnp.float32),
      scratch_shapes=[pltpu.VMEM((1, 128), jnp.float32)],
  )(agg2cat, z2, degp2, W2, b2, Wfc, bfc)


def kernel(x, edge_index, W1, b1, W2, b2, Wfc, bfc):
  src = edge_index[0]
  dst = edge_index[1]
  sf, df = _tc_edge_filter(src.reshape(12800, 125),
                           dst.reshape(12800, 125))
  sf = sf.reshape(NRANGE, EE)
  df = df.reshape(NRANGE, EE)
  xp = jnp.pad(x, ((0, NP - NN), (0, 0)))

  degp = _sc_degree(dst)
  degp2 = degp.reshape(2, NP, 1)
  z1 = _tc_prep(degp2, xp)
  agg1p = _sc_agg16(sf, df, z1)
  z2 = _tc_layer1(agg1p, z1, degp2, W1.astype(jnp.float32),
                  b1.reshape(1, 64))
  z2c = [z2[:, 16 * k:16 * (k + 1)] for k in range(4)]
  agg2p = [_sc_agg16(sf, df, zc) for zc in z2c]
  agg2cat = jnp.concatenate(agg2p, axis=2)
  out = _tc_final(agg2cat, z2, degp2, W2.astype(jnp.float32),
                  b2.reshape(1, 128), Wfc, bfc.reshape(1, 1))
  return out.reshape((1,))
